# Initial kernel scaffold; baseline (speedup 1.0000x reference)
import jax
import jax.numpy as jnp
from jax.experimental import pallas as pl


def kernel(encoding, ix):
    raise NotImplementedError("write your pallas kernel here")


# speedup vs baseline: 2.2191x; 2.2191x over previous
import functools

import jax
import jax.numpy as jnp
from jax import lax
from jax.experimental import pallas as pl
from jax.experimental.pallas import tpu as pltpu
from jax.experimental.pallas import tpu_sc as plsc

NC, NS = 2, 16
NW = NC * NS
CHUNK = 128


def _gather_rows(table, idx_flat, B, D):
    b_per_w = B // NW
    n_chunks = b_per_w // CHUNK
    mesh = plsc.VectorSubcoreMesh(
        core_axis_name="c", subcore_axis_name="s",
        num_cores=NC, num_subcores=NS)

    @functools.partial(
        pl.kernel,
        out_type=jax.ShapeDtypeStruct((B * D,), jnp.float32),
        mesh=mesh,
        scratch_types=[
            pltpu.VMEM((CHUNK,), jnp.int32),
            pltpu.VMEM((CHUNK, 128), jnp.float32),
            pltpu.VMEM((CHUNK * D,), jnp.float32),
            pltpu.SemaphoreType.DMA,
        ],
    )
    def k(table_hbm, idx_hbm, out_hbm, idx_v, rows_v, out_v, gsem):
        wid = lax.axis_index("s") * NC + lax.axis_index("c")
        base = wid * b_per_w

        def body(i, carry):
            off = base + i * CHUNK
            pltpu.sync_copy(idx_hbm.at[pl.ds(off, CHUNK)], idx_v)
            pltpu.async_copy(table_hbm.at[idx_v], rows_v, gsem).wait()

            def crow(r, c):
                for d0 in range(0, D, 16):
                    out_v[pl.ds(r * D + d0, 16)] = rows_v[r, pl.ds(d0, 16)]
                return c

            lax.fori_loop(0, CHUNK, crow, 0, unroll=4)
            pltpu.sync_copy(out_v, out_hbm.at[pl.ds(off * D, CHUNK * D)])
            return carry

        lax.fori_loop(0, n_chunks, body, 0, unroll=False)

    return k(table, idx_flat)


def kernel(encoding, ix):
    B = ix.shape[0] * ix.shape[1]
    D = encoding.shape[1]
    idx_flat = ix.reshape(B).astype(jnp.int32)
    table_padded = jnp.pad(encoding, ((0, 0), (0, 128 - D)))
    out = _gather_rows(table_padded, idx_flat, B, D)
    return out.reshape(ix.shape[0], ix.shape[1], D)


# speedup vs baseline: 3.2545x; 1.4666x over previous
import functools

import jax
import jax.numpy as jnp
from jax import lax
from jax.experimental import pallas as pl
from jax.experimental.pallas import tpu as pltpu
from jax.experimental.pallas import tpu_sc as plsc

NC, NS = 2, 16
NW = NC * NS
CHUNK = 128
NBUF = 2


def _gather_rows(table, idx_flat, B, D):
    b_per_w = B // NW
    n_chunks = b_per_w // CHUNK
    assert n_chunks % NBUF == 0 and n_chunks // NBUF >= 2
    mesh = plsc.VectorSubcoreMesh(
        core_axis_name="c", subcore_axis_name="s",
        num_cores=NC, num_subcores=NS)

    @functools.partial(
        pl.kernel,
        out_type=jax.ShapeDtypeStruct((B * D,), jnp.float32),
        mesh=mesh,
        scratch_types=[
            pltpu.VMEM((b_per_w,), jnp.int32),
            pltpu.VMEM((NBUF, CHUNK, 128), jnp.float32),
            pltpu.VMEM((NBUF, CHUNK * D), jnp.float32),
            [pltpu.SemaphoreType.DMA] * NBUF,
            [pltpu.SemaphoreType.DMA] * NBUF,
        ],
    )
    def k(table_hbm, idx_hbm, out_hbm, idx_v, rows_v, out_v, gsems, osems):
        wid = lax.axis_index("s") * NC + lax.axis_index("c")
        base = wid * b_per_w

        pltpu.sync_copy(idx_hbm.at[pl.ds(base, b_per_w)], idx_v)

        def start_gather(c, s):
            pltpu.async_copy(table_hbm.at[idx_v.at[pl.ds(c * CHUNK, CHUNK)]],
                             rows_v.at[s], gsems[s])

        def wait_gather(c, s):
            pltpu.make_async_copy(
                table_hbm.at[idx_v.at[pl.ds(c * CHUNK, CHUNK)]],
                rows_v.at[s], gsems[s]).wait()

        def compact(s):
            def crow(r, carry):
                for d0 in range(0, D, 16):
                    out_v[s, pl.ds(r * D + d0, 16)] = \
                        rows_v[s, r, pl.ds(d0, 16)]
                return carry
            lax.fori_loop(0, CHUNK, crow, 0, unroll=8)

        def start_store(c, s):
            pltpu.async_copy(
                out_v.at[s],
                out_hbm.at[pl.ds((base + c * CHUNK) * D, CHUNK * D)],
                osems[s])

        def wait_store(c, s):
            pltpu.make_async_copy(
                out_v.at[s],
                out_hbm.at[pl.ds((base + c * CHUNK) * D, CHUNK * D)],
                osems[s]).wait()

        for s in range(NBUF):
            start_gather(s, s)

        for s in range(NBUF):
            wait_gather(s, s)
            compact(s)
            start_store(s, s)
            start_gather(s + NBUF, s)

        def body(j, carry):
            for s in range(NBUF):
                c = j * NBUF + s
                wait_store(c - NBUF, s)
                wait_gather(c, s)
                compact(s)
                start_store(c, s)
                start_gather(c + NBUF, s)
            return carry

        lax.fori_loop(1, n_chunks // NBUF - 1, body, 0, unroll=False)

        for s in range(NBUF):
            c = n_chunks - NBUF + s
            wait_store(c - NBUF, s)
            wait_gather(c, s)
            compact(s)
            start_store(c, s)
        for s in range(NBUF):
            wait_store(n_chunks - NBUF + s, s)

    return k(table, idx_flat)


def kernel(encoding, ix):
    B = ix.shape[0] * ix.shape[1]
    D = encoding.shape[1]
    idx_flat = ix.reshape(B).astype(jnp.int32)
    table_padded = jnp.pad(encoding, ((0, 0), (0, 128 - D)))
    out = _gather_rows(table_padded, idx_flat, B, D)
    return out.reshape(ix.shape[0], ix.shape[1], D)


# speedup vs baseline: 4.2676x; 1.3113x over previous
import functools

import jax
import jax.numpy as jnp
from jax import lax
from jax.experimental import pallas as pl
from jax.experimental.pallas import tpu as pltpu
from jax.experimental.pallas import tpu_sc as plsc

NC, NS = 2, 16
NW = NC * NS
CHUNK = 128
NBUF = 4


def _gather_rows(table, idx_flat, B, D):
    b_per_w = B // NW
    n_chunks = b_per_w // CHUNK
    n_groups = n_chunks // NBUF
    assert n_chunks % NBUF == 0 and n_groups >= 3
    mesh = plsc.VectorSubcoreMesh(
        core_axis_name="c", subcore_axis_name="s",
        num_cores=NC, num_subcores=NS)

    @functools.partial(
        pl.kernel,
        out_type=jax.ShapeDtypeStruct((B, D), jnp.float32),
        mesh=mesh,
        compiler_params=pltpu.CompilerParams(use_tc_tiling_on_sc=False),
        scratch_types=[
            pltpu.VMEM((b_per_w,), jnp.int32),
            pltpu.VMEM((NBUF, CHUNK, D), jnp.float32),
            [pltpu.SemaphoreType.DMA] * NBUF,
            [pltpu.SemaphoreType.DMA] * NBUF,
        ],
    )
    def k(table_hbm, idx_hbm, out_hbm, idx_v, rows_v, gsems, osems):
        wid = lax.axis_index("s") * NC + lax.axis_index("c")
        base = wid * b_per_w

        pltpu.sync_copy(idx_hbm.at[pl.ds(base, b_per_w)], idx_v)

        def start_gather(c, s):
            pltpu.async_copy(table_hbm.at[idx_v.at[pl.ds(c * CHUNK, CHUNK)]],
                             rows_v.at[s], gsems[s])

        def wait_gather(c, s):
            pltpu.make_async_copy(
                table_hbm.at[idx_v.at[pl.ds(c * CHUNK, CHUNK)]],
                rows_v.at[s], gsems[s]).wait()

        def start_store(c, s):
            pltpu.async_copy(rows_v.at[s],
                             out_hbm.at[pl.ds(base + c * CHUNK, CHUNK)],
                             osems[s])

        def wait_store(c, s):
            pltpu.make_async_copy(
                rows_v.at[s],
                out_hbm.at[pl.ds(base + c * CHUNK, CHUNK)],
                osems[s]).wait()

        for s in range(NBUF):
            start_gather(s, s)
        wait_gather(0, 0)
        start_store(0, 0)
        for s in range(1, NBUF):
            wait_store(s - 1, s - 1)
            start_gather(s - 1 + NBUF, s - 1)
            wait_gather(s, s)
            start_store(s, s)

        def body(j, carry):
            for s in range(NBUF):
                c = j * NBUF + s
                sp = (s - 1) % NBUF
                wait_store(c - 1, sp)
                start_gather(c - 1 + NBUF, sp)
                wait_gather(c, s)
                start_store(c, s)
            return carry

        lax.fori_loop(1, n_groups - 1, body, 0, unroll=False)

        for s in range(NBUF):
            c = (n_groups - 1) * NBUF + s
            sp = (s - 1) % NBUF
            wait_store(c - 1, sp)
            if s == 0:
                start_gather(c - 1 + NBUF, sp)
            wait_gather(c, s)
            start_store(c, s)
        wait_store(n_chunks - 1, NBUF - 1)

    return k(table, idx_flat)


def kernel(encoding, ix):
    B = ix.shape[0] * ix.shape[1]
    D = encoding.shape[1]
    idx_flat = ix.reshape(B).astype(jnp.int32)
    out = _gather_rows(encoding, idx_flat, B, D)
    return out.reshape(ix.shape[0], ix.shape[1], D)
